# baseline (speedup 1.0000x reference)
import functools

import jax
import jax.numpy as jnp
from jax import lax
from jax.experimental import pallas as pl
from jax.experimental.pallas import tpu as pltpu
from jax.experimental.pallas import tpu_sc as plsc

NC, NS, L = 2, 16, 16
NW = NC * NS

F32 = jnp.float32


def _mesh():
    return plsc.VectorSubcoreMesh(
        core_axis_name="c", subcore_axis_name="s", num_cores=NC, num_subcores=NS
    )


def _make_gather(V, D, B, C):
    NCH = B // (NW * C)
    assert NCH * NW * C == B and C % 8 == 0 and C <= 128

    @functools.partial(
        pl.kernel,
        out_type=jax.ShapeDtypeStruct((B, D), F32),
        mesh=_mesh(),
        scratch_types=[
            pltpu.VMEM((NCH, C), jnp.int32),
            pltpu.VMEM((C, D), F32),
            pltpu.SemaphoreType.DMA,
        ],
    )
    def k(table_hbm, idx_hbm, out_hbm, idx_v, rows_v, sem):
        wid = lax.axis_index("c") * NS + lax.axis_index("s")
        pltpu.sync_copy(idx_hbm.at[wid], idx_v)

        def body(j, carry):
            pltpu.async_copy(table_hbm.at[idx_v.at[j]], rows_v, sem).wait()
            pltpu.sync_copy(rows_v, out_hbm.at[pl.ds((wid * NCH + j) * C, C)])
            return carry

        lax.fori_loop(0, NCH, body, 0)

    return k


def _make_segsum(Vt, Vp, E, C, NCH0, NCH1):
    NB = 8
    RPT = Vp // NS
    assert (NCH0 + NCH1) * NS * C == E and RPT * NS == Vp
    assert C % 8 == 0 and C <= 128 and NB % 2 == 0
    assert NCH0 % NB == 0 and NCH1 % NB == 0

    @functools.partial(
        pl.kernel,
        out_type=jax.ShapeDtypeStruct((NC, Vp, 128), F32),
        mesh=_mesh(),
        scratch_types=[
            pltpu.VMEM((NB, C), jnp.int32),
            pltpu.VMEM((NB, C), jnp.int32),
            pltpu.VMEM((C, 128), F32),
            pltpu.VMEM((C, 128), F32),
            pltpu.VMEM_SHARED((Vp, 128), F32),
            pltpu.SemaphoreType.DMA,
            pltpu.SemaphoreType.DMA,
        ],
    )
    def k(h_hbm, src_hbm, dst_hbm, z128_hbm, agg_hbm,
          src_v, dst_v, rows0_v, rows1_v, acc_sh, sem0, sem1):
        c = lax.axis_index("c")
        s = lax.axis_index("s")
        nch = jnp.where(c == 0, NCH0, NCH1)
        base = c * NS * NCH0 + s * nch
        pltpu.sync_copy(z128_hbm, acc_sh.at[pl.ds(s * RPT, RPT)])
        plsc.subcore_barrier()

        def group(g, carry):
            pltpu.sync_copy(src_hbm.at[pl.ds(base + g * NB, NB)], src_v)
            pltpu.sync_copy(dst_hbm.at[pl.ds(base + g * NB, NB)], dst_v)
            d0 = pltpu.async_copy(h_hbm.at[src_v.at[0]], rows0_v, sem0)

            def pair(t, carry2):
                pltpu.async_copy(h_hbm.at[src_v.at[2 * t + 1]], rows1_v, sem1)
                d0.wait()
                pltpu.sync_copy(rows0_v, acc_sh.at[dst_v.at[2 * t]], add=True)

                @pl.when(t < NB // 2 - 1)
                def _():
                    pltpu.async_copy(h_hbm.at[src_v.at[2 * t + 2]], rows0_v,
                                     sem0)

                d1 = pltpu.make_async_copy(h_hbm.at[src_v.at[0]], rows1_v,
                                           sem1)
                d1.wait()
                pltpu.sync_copy(rows1_v, acc_sh.at[dst_v.at[2 * t + 1]],
                                add=True)
                return carry2

            lax.fori_loop(0, NB // 2, pair, carry)
            return carry

        lax.fori_loop(0, nch // NB, group, 0)
        plsc.subcore_barrier()
        pltpu.sync_copy(acc_sh.at[pl.ds(s * RPT, RPT)],
                        agg_hbm.at[c].at[pl.ds(s * RPT, RPT)])

    return k


def _make_cnt(Vp, E, C):
    NB = 16
    NCH = E // (NW * C)
    NG = NCH // NB
    RPT = Vp // NS
    assert NCH * NW * C == E and RPT * NS == Vp and NG * NB == NCH

    @functools.partial(
        pl.kernel,
        out_type=jax.ShapeDtypeStruct((NC, Vp, 128), F32),
        mesh=_mesh(),
        scratch_types=[
            pltpu.VMEM((NB, C), jnp.int32),
            pltpu.VMEM((C, 128), F32),
            pltpu.VMEM_SHARED((Vp, 128), F32),
        ],
    )
    def k(dst_hbm, z128_hbm, ones_hbm, cnt_hbm, dst_v, ones_v, cac_sh):
        c = lax.axis_index("c")
        s = lax.axis_index("s")
        wid = c * NS + s
        pltpu.sync_copy(ones_hbm, ones_v)
        pltpu.sync_copy(z128_hbm, cac_sh.at[pl.ds(s * RPT, RPT)])
        plsc.subcore_barrier()

        def group(g, carry):
            pltpu.sync_copy(dst_hbm.at[pl.ds(wid * NCH + g * NB, NB)], dst_v)

            def body(j, carry2):
                pltpu.sync_copy(ones_v, cac_sh.at[dst_v.at[j]], add=True)
                return carry2

            lax.fori_loop(0, NB, body, carry)
            return carry

        lax.fori_loop(0, NG, group, 0)
        plsc.subcore_barrier()
        pltpu.sync_copy(cac_sh.at[pl.ds(s * RPT, RPT)],
                        cnt_hbm.at[c].at[pl.ds(s * RPT, RPT)])

    return k


def _layer1(aggp, cntp, h0, wl_t, wr_t, b):
    Np = h0.shape[0]
    BR = 512

    def body(aggp_ref, cntp_ref, h0_ref, wl_ref, wr_ref, b_ref,
             out_ref, rcp_ref):
        agg = aggp_ref[0] + aggp_ref[1]
        cnt = cntp_ref[0, :, 0] + cntp_ref[1, :, 0]
        rcp = 1.0 / jnp.maximum(cnt, 1.0)
        mean = agg * rcp[:, None]
        h = (jnp.dot(mean, wl_ref[...], preferred_element_type=F32)
             + jnp.dot(h0_ref[...], wr_ref[...], preferred_element_type=F32)
             + b_ref[...])
        out_ref[...] = jnp.maximum(h, 0.0)
        rcp_ref[...] = jnp.broadcast_to(rcp[:, None], (BR, 128))

    return pl.pallas_call(
        body,
        grid=(Np // BR,),
        in_specs=[
            pl.BlockSpec((NC, BR, 128), lambda i: (0, i, 0)),
            pl.BlockSpec((NC, BR, 128), lambda i: (0, i, 0)),
            pl.BlockSpec((BR, 128), lambda i: (i, 0)),
            pl.BlockSpec((128, 128), lambda i: (0, 0)),
            pl.BlockSpec((128, 128), lambda i: (0, 0)),
            pl.BlockSpec((1, 128), lambda i: (0, 0)),
        ],
        out_specs=[pl.BlockSpec((BR, 128), lambda i: (i, 0)),
                   pl.BlockSpec((BR, 128), lambda i: (i, 0))],
        out_shape=[jax.ShapeDtypeStruct((Np, 128), F32),
                   jax.ShapeDtypeStruct((Np, 128), F32)],
    )(aggp, cntp, h0, wl_t, wr_t, b)


def _layer2(agg0, agg1, rcpm, h1m, wl_t, wr_t, b):
    B, Kp = h1m.shape[0], wl_t.shape[1]
    BR = 256

    def body(a0_ref, a1_ref, r_ref, h_ref, wl_ref, wr_ref, b_ref, out_ref):
        mean = (a0_ref[...] + a1_ref[...]) * r_ref[...]
        logits = (jnp.dot(mean, wl_ref[...], preferred_element_type=F32)
                  + jnp.dot(h_ref[...], wr_ref[...], preferred_element_type=F32)
                  + b_ref[...])
        m = jnp.max(logits, axis=1, keepdims=True)
        z = logits - m
        lse = jnp.log(jnp.sum(jnp.exp(z), axis=1, keepdims=True))
        out_ref[...] = z - lse

    return pl.pallas_call(
        body,
        grid=(B // BR,),
        in_specs=[
            pl.BlockSpec((BR, 128), lambda i: (i, 0)),
            pl.BlockSpec((BR, 128), lambda i: (i, 0)),
            pl.BlockSpec((BR, 128), lambda i: (i, 0)),
            pl.BlockSpec((BR, 128), lambda i: (i, 0)),
            pl.BlockSpec((128, Kp), lambda i: (0, 0)),
            pl.BlockSpec((128, Kp), lambda i: (0, 0)),
            pl.BlockSpec((1, Kp), lambda i: (0, 0)),
        ],
        out_specs=pl.BlockSpec((BR, Kp), lambda i: (i, 0)),
        out_shape=jax.ShapeDtypeStruct((B, Kp), F32),
    )(agg0, agg1, rcpm, h1m, wl_t, wr_t, b)


def kernel(x, edge_index, mask_x_position, emb, W1_l, W1_r, b1, W2_l, W2_r, b2):
    V, D = emb.shape
    N = x.shape[0]
    E = edge_index.shape[1]
    M = mask_x_position.shape[0]
    K = W2_l.shape[0]
    Np = 10240
    Kp = 10240
    CE = 128
    Ep = 327680
    CG = 80

    xi = x[:, 0].astype(jnp.int32)
    xi_pad = jnp.concatenate([xi, jnp.zeros((Np - N,), jnp.int32)])
    src_pad = jnp.concatenate(
        [edge_index[0].astype(jnp.int32), jnp.zeros((Ep - E,), jnp.int32)])
    dst_pad = jnp.concatenate(
        [edge_index[1].astype(jnp.int32),
         jnp.full((Ep - E,), N, jnp.int32)])
    src2d = src_pad.reshape(Ep // CE, CE)
    dst2d = dst_pad.reshape(Ep // CE, CE)
    mask = mask_x_position.astype(jnp.int32)

    z128 = jnp.zeros((Np // NS, 128), F32)
    ones = jnp.ones((CE, 128), F32)

    NCH0, NCH1 = 160, 0

    gather_nodes = _make_gather(V, D, Np, CG)
    segsum = _make_segsum(Np, Np, Ep, CE, NCH0, NCH1)
    cntk = _make_cnt(Np, Ep, CE)
    gather_m = _make_gather(Np, D, M, M // NW)
    gather_m2 = _make_gather(NC * Np, D, NC * M, NC * M // NW)

    h0 = gather_nodes(emb, xi_pad.reshape(NW, Np // (NW * CG), CG))
    cntp1 = cntk(dst2d, z128, ones)

    aggp1 = segsum(h0, src2d, dst2d, z128)
    h1, rcpb = _layer1(aggp1, cntp1, h0, W1_l.T, W1_r.T, b1.reshape(1, 128))

    aggp2 = segsum(h1, src2d, dst2d, z128)

    mask2 = jnp.concatenate([mask, mask + Np])
    h1m = gather_m(h1, mask.reshape(NW, 1, M // NW))
    rcpm = gather_m(rcpb, mask.reshape(NW, 1, M // NW))
    aggm = gather_m2(aggp2.reshape(NC * Np, D),
                     mask2.reshape(NW, 1, NC * M // NW))

    wl_t = jnp.zeros((128, Kp), F32).at[:, :K].set(W2_l.T)
    wr_t = jnp.zeros((128, Kp), F32).at[:, :K].set(W2_r.T)
    b2p = jnp.full((1, Kp), -1e30, F32).at[0, :K].set(b2)
    out = _layer2(aggm[:M], aggm[M:], rcpm, h1m, wl_t, wr_t, b2p)
    return out[:, :K]


# speedup vs baseline: 1.3805x; 1.3805x over previous
import functools

import jax
import jax.numpy as jnp
from jax import lax
from jax.experimental import pallas as pl
from jax.experimental.pallas import tpu as pltpu
from jax.experimental.pallas import tpu_sc as plsc

NC, NS, L = 2, 16, 16
NW = NC * NS

F32 = jnp.float32


def _mesh():
    return plsc.VectorSubcoreMesh(
        core_axis_name="c", subcore_axis_name="s", num_cores=NC, num_subcores=NS
    )


def _make_gather(V, D, B, C):
    NCH = B // (NW * C)
    assert NCH * NW * C == B and C % 8 == 0 and C <= 128

    @functools.partial(
        pl.kernel,
        out_type=jax.ShapeDtypeStruct((B, D), F32),
        mesh=_mesh(),
        scratch_types=[
            pltpu.VMEM((NCH, C), jnp.int32),
            pltpu.VMEM((C, D), F32),
            pltpu.SemaphoreType.DMA,
        ],
    )
    def k(table_hbm, idx_hbm, out_hbm, idx_v, rows_v, sem):
        wid = lax.axis_index("c") * NS + lax.axis_index("s")
        pltpu.sync_copy(idx_hbm.at[wid], idx_v)

        def body(j, carry):
            pltpu.async_copy(table_hbm.at[idx_v.at[j]], rows_v, sem).wait()
            pltpu.sync_copy(rows_v, out_hbm.at[pl.ds((wid * NCH + j) * C, C)])
            return carry

        lax.fori_loop(0, NCH, body, 0)

    return k


def _make_segsum(Vt, Vp, E, C, NCH0, NCH1):
    NB = 8
    RPT = Vp // NS
    assert (NCH0 + NCH1) * NS * C == E and RPT * NS == Vp
    assert C % 8 == 0 and C <= 128 and NB % 2 == 0
    assert NCH0 % NB == 0 and NCH1 % NB == 0

    @functools.partial(
        pl.kernel,
        out_type=jax.ShapeDtypeStruct((NC, Vp, 128), F32),
        mesh=_mesh(),
        scratch_types=[
            pltpu.VMEM((NB, C), jnp.int32),
            pltpu.VMEM((NB, C), jnp.int32),
            pltpu.VMEM((C, 128), F32),
            pltpu.VMEM((C, 128), F32),
            pltpu.VMEM_SHARED((Vp, 128), F32),
            pltpu.SemaphoreType.DMA,
            pltpu.SemaphoreType.DMA,
        ],
    )
    def k(h_hbm, src_hbm, dst_hbm, z128_hbm, agg_hbm,
          src_v, dst_v, rows0_v, rows1_v, acc_sh, sem0, sem1):
        c = lax.axis_index("c")
        s = lax.axis_index("s")
        nch = jnp.where(c == 0, NCH0, NCH1)
        base = c * NS * NCH0 + s * nch
        pltpu.sync_copy(z128_hbm, acc_sh.at[pl.ds(s * RPT, RPT)])
        plsc.subcore_barrier()

        def group(g, carry):
            pltpu.sync_copy(src_hbm.at[pl.ds(base + g * NB, NB)], src_v)
            pltpu.sync_copy(dst_hbm.at[pl.ds(base + g * NB, NB)], dst_v)
            d0 = pltpu.async_copy(h_hbm.at[src_v.at[0]], rows0_v, sem0)

            def pair(t, carry2):
                pltpu.async_copy(h_hbm.at[src_v.at[2 * t + 1]], rows1_v, sem1)
                d0.wait()
                pltpu.sync_copy(rows0_v, acc_sh.at[dst_v.at[2 * t]], add=True)

                @pl.when(t < NB // 2 - 1)
                def _():
                    pltpu.async_copy(h_hbm.at[src_v.at[2 * t + 2]], rows0_v,
                                     sem0)

                d1 = pltpu.make_async_copy(h_hbm.at[src_v.at[0]], rows1_v,
                                           sem1)
                d1.wait()
                pltpu.sync_copy(rows1_v, acc_sh.at[dst_v.at[2 * t + 1]],
                                add=True)
                return carry2

            lax.fori_loop(0, NB // 2, pair, carry)
            return carry

        lax.fori_loop(0, nch // NB, group, 0)
        plsc.subcore_barrier()
        pltpu.sync_copy(acc_sh.at[pl.ds(s * RPT, RPT)],
                        agg_hbm.at[c].at[pl.ds(s * RPT, RPT)])

    return k


def _make_cnt(Vp, E, C):
    NB = 16
    NCH = E // (NW * C)
    NG = NCH // NB
    RPT = Vp // NS
    assert NCH * NW * C == E and RPT * NS == Vp and NG * NB == NCH

    @functools.partial(
        pl.kernel,
        out_type=jax.ShapeDtypeStruct((NC, Vp, 128), F32),
        mesh=_mesh(),
        scratch_types=[
            pltpu.VMEM((NB, C), jnp.int32),
            pltpu.VMEM((C, 128), F32),
            pltpu.VMEM_SHARED((Vp, 128), F32),
        ],
    )
    def k(dst_hbm, z128_hbm, ones_hbm, cnt_hbm, dst_v, ones_v, cac_sh):
        c = lax.axis_index("c")
        s = lax.axis_index("s")
        wid = c * NS + s
        pltpu.sync_copy(ones_hbm, ones_v)
        pltpu.sync_copy(z128_hbm, cac_sh.at[pl.ds(s * RPT, RPT)])
        plsc.subcore_barrier()

        def group(g, carry):
            pltpu.sync_copy(dst_hbm.at[pl.ds(wid * NCH + g * NB, NB)], dst_v)

            def body(j, carry2):
                pltpu.sync_copy(ones_v, cac_sh.at[dst_v.at[j]], add=True)
                return carry2

            lax.fori_loop(0, NB, body, carry)
            return carry

        lax.fori_loop(0, NG, group, 0)
        plsc.subcore_barrier()
        pltpu.sync_copy(cac_sh.at[pl.ds(s * RPT, RPT)],
                        cnt_hbm.at[c].at[pl.ds(s * RPT, RPT)])

    return k


def _layer1(aggp, cntp, h0, wl_t, wr_t, b):
    Np = h0.shape[0]
    BR = 512

    def body(aggp_ref, cntp_ref, h0_ref, wl_ref, wr_ref, b_ref,
             out_ref, rcp_ref):
        agg = aggp_ref[0] + aggp_ref[1]
        cnt = cntp_ref[0, :, 0] + cntp_ref[1, :, 0]
        rcp = 1.0 / jnp.maximum(cnt, 1.0)
        mean = agg * rcp[:, None]
        h = (jnp.dot(mean, wl_ref[...], preferred_element_type=F32)
             + jnp.dot(h0_ref[...], wr_ref[...], preferred_element_type=F32)
             + b_ref[...])
        out_ref[...] = jnp.maximum(h, 0.0)
        rcp_ref[...] = jnp.broadcast_to(rcp[:, None], (BR, 128))

    return pl.pallas_call(
        body,
        grid=(Np // BR,),
        in_specs=[
            pl.BlockSpec((NC, BR, 128), lambda i: (0, i, 0)),
            pl.BlockSpec((NC, BR, 128), lambda i: (0, i, 0)),
            pl.BlockSpec((BR, 128), lambda i: (i, 0)),
            pl.BlockSpec((128, 128), lambda i: (0, 0)),
            pl.BlockSpec((128, 128), lambda i: (0, 0)),
            pl.BlockSpec((1, 128), lambda i: (0, 0)),
        ],
        out_specs=[pl.BlockSpec((BR, 128), lambda i: (i, 0)),
                   pl.BlockSpec((BR, 128), lambda i: (i, 0))],
        out_shape=[jax.ShapeDtypeStruct((Np, 128), F32),
                   jax.ShapeDtypeStruct((Np, 128), F32)],
    )(aggp, cntp, h0, wl_t, wr_t, b)


def _layer2(agg0, agg1, rcpm, h1m, wl_t, wr_t, b):
    B, Kp = h1m.shape[0], wl_t.shape[1]
    BR = 256

    def body(a0_ref, a1_ref, r_ref, h_ref, wl_ref, wr_ref, b_ref, out_ref):
        mean = (a0_ref[...] + a1_ref[...]) * r_ref[...]
        logits = (jnp.dot(mean, wl_ref[...], preferred_element_type=F32)
                  + jnp.dot(h_ref[...], wr_ref[...], preferred_element_type=F32)
                  + b_ref[...])
        m = jnp.max(logits, axis=1, keepdims=True)
        z = logits - m
        lse = jnp.log(jnp.sum(jnp.exp(z), axis=1, keepdims=True))
        out_ref[...] = z - lse

    return pl.pallas_call(
        body,
        grid=(B // BR,),
        in_specs=[
            pl.BlockSpec((BR, 128), lambda i: (i, 0)),
            pl.BlockSpec((BR, 128), lambda i: (i, 0)),
            pl.BlockSpec((BR, 128), lambda i: (i, 0)),
            pl.BlockSpec((BR, 128), lambda i: (i, 0)),
            pl.BlockSpec((128, Kp), lambda i: (0, 0)),
            pl.BlockSpec((128, Kp), lambda i: (0, 0)),
            pl.BlockSpec((1, Kp), lambda i: (0, 0)),
        ],
        out_specs=pl.BlockSpec((BR, Kp), lambda i: (i, 0)),
        out_shape=jax.ShapeDtypeStruct((B, Kp), F32),
    )(agg0, agg1, rcpm, h1m, wl_t, wr_t, b)


def kernel(x, edge_index, mask_x_position, emb, W1_l, W1_r, b1, W2_l, W2_r, b2):
    V, D = emb.shape
    N = x.shape[0]
    E = edge_index.shape[1]
    M = mask_x_position.shape[0]
    K = W2_l.shape[0]
    Np = 10240
    Kp = 10240
    CE = 128
    Ep = 327680
    CG = 80

    xi = x[:, 0].astype(jnp.int32)
    xi_pad = jnp.concatenate([xi, jnp.zeros((Np - N,), jnp.int32)])
    src_pad = jnp.concatenate(
        [edge_index[0].astype(jnp.int32), jnp.zeros((Ep - E,), jnp.int32)])
    dst_pad = jnp.concatenate(
        [edge_index[1].astype(jnp.int32),
         jnp.full((Ep - E,), N, jnp.int32)])
    src2d = src_pad.reshape(Ep // CE, CE)
    dst2d = dst_pad.reshape(Ep // CE, CE)
    mask = mask_x_position.astype(jnp.int32)

    z128 = jnp.zeros((Np // NS, 128), F32)
    ones = jnp.ones((CE, 128), F32)

    NCH0, NCH1 = 152, 8

    gather_nodes = _make_gather(V, D, Np, CG)
    segsum = _make_segsum(Np, Np, Ep, CE, NCH0, NCH1)
    cntk = _make_cnt(Np, Ep, CE)
    gather_m = _make_gather(Np, D, M, M // NW)
    gather_m2 = _make_gather(NC * Np, D, NC * M, NC * M // NW)

    h0 = gather_nodes(emb, xi_pad.reshape(NW, Np // (NW * CG), CG))
    cntp1 = cntk(dst2d, z128, ones)

    aggp1 = segsum(h0, src2d, dst2d, z128)
    h1, rcpb = _layer1(aggp1, cntp1, h0, W1_l.T, W1_r.T, b1.reshape(1, 128))

    aggp2 = segsum(h1, src2d, dst2d, z128)

    mask2 = jnp.concatenate([mask, mask + Np])
    h1m = gather_m(h1, mask.reshape(NW, 1, M // NW))
    rcpm = gather_m(rcpb, mask.reshape(NW, 1, M // NW))
    aggm = gather_m2(aggp2.reshape(NC * Np, D),
                     mask2.reshape(NW, 1, NC * M // NW))

    wl_t = jnp.zeros((128, Kp), F32).at[:, :K].set(W2_l.T)
    wr_t = jnp.zeros((128, Kp), F32).at[:, :K].set(W2_r.T)
    b2p = jnp.full((1, Kp), -1e30, F32).at[0, :K].set(b2)
    out = _layer2(aggm[:M], aggm[M:], rcpm, h1m, wl_t, wr_t, b2p)
    return out[:, :K]


# speedup vs baseline: 1.4410x; 1.0438x over previous
import functools

import jax
import jax.numpy as jnp
from jax import lax
from jax.experimental import pallas as pl
from jax.experimental.pallas import tpu as pltpu
from jax.experimental.pallas import tpu_sc as plsc

NC, NS, L = 2, 16, 16
NW = NC * NS

F32 = jnp.float32


def _mesh():
    return plsc.VectorSubcoreMesh(
        core_axis_name="c", subcore_axis_name="s", num_cores=NC, num_subcores=NS
    )


def _make_gather(V, D, B, C):
    NCH = B // (NW * C)
    assert NCH * NW * C == B and C % 8 == 0 and C <= 128

    @functools.partial(
        pl.kernel,
        out_type=jax.ShapeDtypeStruct((B, D), F32),
        mesh=_mesh(),
        scratch_types=[
            pltpu.VMEM((NCH, C), jnp.int32),
            pltpu.VMEM((C, D), F32),
            pltpu.SemaphoreType.DMA,
        ],
    )
    def k(table_hbm, idx_hbm, out_hbm, idx_v, rows_v, sem):
        wid = lax.axis_index("c") * NS + lax.axis_index("s")
        pltpu.sync_copy(idx_hbm.at[wid], idx_v)

        def body(j, carry):
            pltpu.async_copy(table_hbm.at[idx_v.at[j]], rows_v, sem).wait()
            pltpu.sync_copy(rows_v, out_hbm.at[pl.ds((wid * NCH + j) * C, C)])
            return carry

        lax.fori_loop(0, NCH, body, 0)

    return k


def _make_segsum(Vt, Vp, E, C, NCH0, NCH1):
    NB = 8
    RPT = Vp // NS
    assert (NCH0 + NCH1) * NS * C == E and RPT * NS == Vp
    assert C % 8 == 0 and C <= 128 and NB % 2 == 0
    assert NCH0 % NB == 0 and NCH1 % NB == 0

    @functools.partial(
        pl.kernel,
        out_type=jax.ShapeDtypeStruct((NC, Vp, 128), F32),
        mesh=_mesh(),
        scratch_types=[
            pltpu.VMEM((NB, C), jnp.int32),
            pltpu.VMEM((NB, C), jnp.int32),
            pltpu.VMEM((C, 128), F32),
            pltpu.VMEM((C, 128), F32),
            pltpu.VMEM_SHARED((Vp, 128), F32),
            pltpu.SemaphoreType.DMA,
            pltpu.SemaphoreType.DMA,
        ],
    )
    def k(h_hbm, src_hbm, dst_hbm, z128_hbm, agg_hbm,
          src_v, dst_v, rows0_v, rows1_v, acc_sh, sem0, sem1):
        c = lax.axis_index("c")
        s = lax.axis_index("s")
        nch = jnp.where(c == 0, NCH0, NCH1)
        base = c * NS * NCH0 + s * nch
        pltpu.sync_copy(z128_hbm, acc_sh.at[pl.ds(s * RPT, RPT)])
        plsc.subcore_barrier()

        def group(g, carry):
            pltpu.sync_copy(src_hbm.at[pl.ds(base + g * NB, NB)], src_v)
            pltpu.sync_copy(dst_hbm.at[pl.ds(base + g * NB, NB)], dst_v)
            d0 = pltpu.async_copy(h_hbm.at[src_v.at[0]], rows0_v, sem0)

            def pair(t, carry2):
                pltpu.async_copy(h_hbm.at[src_v.at[2 * t + 1]], rows1_v, sem1)
                d0.wait()
                pltpu.sync_copy(rows0_v, acc_sh.at[dst_v.at[2 * t]], add=True)

                @pl.when(t < NB // 2 - 1)
                def _():
                    pltpu.async_copy(h_hbm.at[src_v.at[2 * t + 2]], rows0_v,
                                     sem0)

                d1 = pltpu.make_async_copy(h_hbm.at[src_v.at[0]], rows1_v,
                                           sem1)
                d1.wait()
                pltpu.sync_copy(rows1_v, acc_sh.at[dst_v.at[2 * t + 1]],
                                add=True)
                return carry2

            lax.fori_loop(0, NB // 2, pair, carry)
            return carry

        lax.fori_loop(0, nch // NB, group, 0)
        plsc.subcore_barrier()
        pltpu.sync_copy(acc_sh.at[pl.ds(s * RPT, RPT)],
                        agg_hbm.at[c].at[pl.ds(s * RPT, RPT)])

    return k


def _make_cnt(Vp, E, C):
    NB = 16
    NCH = E // (NW * C)
    NG = NCH // NB
    RPT = Vp // NS
    assert NCH * NW * C == E and RPT * NS == Vp and NG * NB == NCH

    @functools.partial(
        pl.kernel,
        out_type=jax.ShapeDtypeStruct((NC, Vp, 128), F32),
        mesh=_mesh(),
        scratch_types=[
            pltpu.VMEM((NB, C), jnp.int32),
            pltpu.VMEM((C, 128), F32),
            pltpu.VMEM_SHARED((Vp, 128), F32),
        ],
    )
    def k(dst_hbm, z128_hbm, ones_hbm, cnt_hbm, dst_v, ones_v, cac_sh):
        c = lax.axis_index("c")
        s = lax.axis_index("s")
        wid = c * NS + s
        pltpu.sync_copy(ones_hbm, ones_v)
        pltpu.sync_copy(z128_hbm, cac_sh.at[pl.ds(s * RPT, RPT)])
        plsc.subcore_barrier()

        def group(g, carry):
            pltpu.sync_copy(dst_hbm.at[pl.ds(wid * NCH + g * NB, NB)], dst_v)

            def body(j, carry2):
                pltpu.sync_copy(ones_v, cac_sh.at[dst_v.at[j]], add=True)
                return carry2

            lax.fori_loop(0, NB, body, carry)
            return carry

        lax.fori_loop(0, NG, group, 0)
        plsc.subcore_barrier()
        pltpu.sync_copy(cac_sh.at[pl.ds(s * RPT, RPT)],
                        cnt_hbm.at[c].at[pl.ds(s * RPT, RPT)])

    return k


def _layer1(aggp, cntp, h0, wl_t, wr_t, b):
    Np = h0.shape[0]
    BR = 512

    def body(aggp_ref, cntp_ref, h0_ref, wl_ref, wr_ref, b_ref, out_ref):
        agg = aggp_ref[0] + aggp_ref[1]
        cnt = cntp_ref[0, :, 0] + cntp_ref[1, :, 0]
        rcp = 1.0 / jnp.maximum(cnt, 1.0)
        mean = agg * rcp[:, None]
        h = (jnp.dot(mean, wl_ref[...], preferred_element_type=F32)
             + jnp.dot(h0_ref[...], wr_ref[...], preferred_element_type=F32)
             + b_ref[...])
        out_ref[0] = jnp.maximum(h, 0.0)
        out_ref[1] = jnp.broadcast_to(rcp[:, None], (BR, 128))

    return pl.pallas_call(
        body,
        grid=(Np // BR,),
        in_specs=[
            pl.BlockSpec((NC, BR, 128), lambda i: (0, i, 0)),
            pl.BlockSpec((NC, BR, 128), lambda i: (0, i, 0)),
            pl.BlockSpec((BR, 128), lambda i: (i, 0)),
            pl.BlockSpec((128, 128), lambda i: (0, 0)),
            pl.BlockSpec((128, 128), lambda i: (0, 0)),
            pl.BlockSpec((1, 128), lambda i: (0, 0)),
        ],
        out_specs=pl.BlockSpec((2, BR, 128), lambda i: (0, i, 0)),
        out_shape=jax.ShapeDtypeStruct((2, Np, 128), F32),
    )(aggp, cntp, h0, wl_t, wr_t, b)


def _layer2(agg0, agg1, rcpm, h1m, wl_t, wr_t, b):
    B, Kp = h1m.shape[0], wl_t.shape[1]
    BR = 256

    def body(a0_ref, a1_ref, r_ref, h_ref, wl_ref, wr_ref, b_ref, out_ref):
        mean = (a0_ref[...] + a1_ref[...]) * r_ref[...]
        logits = (jnp.dot(mean, wl_ref[...], preferred_element_type=F32)
                  + jnp.dot(h_ref[...], wr_ref[...], preferred_element_type=F32)
                  + b_ref[...])
        m = jnp.max(logits, axis=1, keepdims=True)
        z = logits - m
        lse = jnp.log(jnp.sum(jnp.exp(z), axis=1, keepdims=True))
        out_ref[...] = z - lse

    return pl.pallas_call(
        body,
        grid=(B // BR,),
        in_specs=[
            pl.BlockSpec((BR, 128), lambda i: (i, 0)),
            pl.BlockSpec((BR, 128), lambda i: (i, 0)),
            pl.BlockSpec((BR, 128), lambda i: (i, 0)),
            pl.BlockSpec((BR, 128), lambda i: (i, 0)),
            pl.BlockSpec((128, Kp), lambda i: (0, 0)),
            pl.BlockSpec((128, Kp), lambda i: (0, 0)),
            pl.BlockSpec((1, Kp), lambda i: (0, 0)),
        ],
        out_specs=pl.BlockSpec((BR, Kp), lambda i: (i, 0)),
        out_shape=jax.ShapeDtypeStruct((B, Kp), F32),
    )(agg0, agg1, rcpm, h1m, wl_t, wr_t, b)


def kernel(x, edge_index, mask_x_position, emb, W1_l, W1_r, b1, W2_l, W2_r, b2):
    V, D = emb.shape
    N = x.shape[0]
    E = edge_index.shape[1]
    M = mask_x_position.shape[0]
    K = W2_l.shape[0]
    Np = 10240
    Kp = 10240
    CE = 128
    Ep = 327680
    CG = 80

    xi = x[:, 0].astype(jnp.int32)
    xi_pad = jnp.concatenate([xi, jnp.zeros((Np - N,), jnp.int32)])
    src_pad = jnp.concatenate(
        [edge_index[0].astype(jnp.int32), jnp.zeros((Ep - E,), jnp.int32)])
    dst_pad = jnp.concatenate(
        [edge_index[1].astype(jnp.int32),
         jnp.full((Ep - E,), N, jnp.int32)])
    src2d = src_pad.reshape(Ep // CE, CE)
    dst2d = dst_pad.reshape(Ep // CE, CE)
    mask = mask_x_position.astype(jnp.int32)

    z128 = jnp.zeros((Np // NS, 128), F32)
    ones = jnp.ones((CE, 128), F32)

    NCH0, NCH1 = 152, 8

    gather_nodes = _make_gather(V, D, Np, CG)
    segsum = _make_segsum(Np, Np, Ep, CE, NCH0, NCH1)
    cntk = _make_cnt(Np, Ep, CE)
    gather_m2 = _make_gather(NC * Np, D, NC * M, NC * M // NW)

    h0 = gather_nodes(emb, xi_pad.reshape(NW, Np // (NW * CG), CG))
    cntp1 = cntk(dst2d, z128, ones)

    aggp1 = segsum(h0, src2d, dst2d, z128)
    hr = _layer1(aggp1, cntp1, h0, W1_l.T, W1_r.T, b1.reshape(1, 128))
    hr2d = hr.reshape(2 * Np, D)

    aggp2 = segsum(hr2d, src2d, dst2d, z128)

    mask2 = jnp.concatenate([mask, mask + Np])
    hm = gather_m2(hr2d, mask2.reshape(NW, 1, NC * M // NW))
    aggm = gather_m2(aggp2.reshape(NC * Np, D),
                     mask2.reshape(NW, 1, NC * M // NW))

    out = _layer2(aggm[:M], aggm[M:], hm[M:], hm[:M],
                  W2_l.T, W2_r.T, b2.reshape(1, K))
    return out
